# baseline (speedup 1.0000x reference)
import functools

import jax
import jax.numpy as jnp
from jax import lax
from jax.experimental import pallas as pl
from jax.experimental.pallas import tpu as pltpu
from jax.experimental.pallas import tpu_sc as plsc

_VOCAB = 1000000
_EMBED = 64
_B = 1024
_L = 512
_NW = 32
_SEQ_PER_W = _B // _NW
_IDX_MINOR = 128
_IDX_ROWS = _L // _IDX_MINOR

_mesh = plsc.VectorSubcoreMesh(core_axis_name="c", subcore_axis_name="s")


@functools.partial(
    pl.kernel,
    out_type=jax.ShapeDtypeStruct((_B, _L, _EMBED), jnp.float32),
    mesh=_mesh,
    scratch_types=[
        pltpu.VMEM((_IDX_ROWS, _IDX_MINOR), jnp.int32),
        pltpu.VMEM((_L, _EMBED), jnp.float32),
        pltpu.VMEM((_L, _EMBED), jnp.float32),
        pltpu.SemaphoreType.DMA,
    ],
    compiler_params=pltpu.CompilerParams(use_tc_tiling_on_sc=False),
)
def _emb_lookup(ids_hbm, table_hbm, pos_hbm, out_hbm, idx_v, rows_v, pos_v, gsem):
    wid = lax.axis_index("s") * 2 + lax.axis_index("c")
    pltpu.sync_copy(pos_hbm, pos_v)

    def seq_body(c, carry):
        b = wid * _SEQ_PER_W + c
        pltpu.sync_copy(ids_hbm.at[b], idx_v)
        cps = [
            pltpu.async_copy(
                table_hbm.at[idx_v.at[j]],
                rows_v.at[pl.ds(j * _IDX_MINOR, _IDX_MINOR)],
                gsem,
            )
            for j in range(_IDX_ROWS)
        ]
        for cp in cps:
            cp.wait()

        def add_body(i, carry2):
            for k in range(_EMBED // 16):
                sl = pl.ds(k * 16, 16)
                rows_v[i, sl] = rows_v[i, sl] + pos_v[i, sl]
            return carry2

        lax.fori_loop(0, _L, add_body, 0)
        pltpu.sync_copy(rows_v, out_hbm.at[b])
        return carry

    lax.fori_loop(0, _SEQ_PER_W, seq_body, 0)


def kernel(input_ids, word_table, pos_table):
    ids3d = input_ids.reshape(_B, _IDX_ROWS, _IDX_MINOR)
    return _emb_lookup(ids3d, word_table, pos_table)
